# baseline (speedup 1.0000x reference)
import functools

import jax
import jax.numpy as jnp
from jax import lax
from jax.experimental import pallas as pl
from jax.experimental.pallas import tpu as pltpu
from jax.experimental.pallas import tpu_sc as plsc

F32 = jnp.float32
I32 = jnp.int32

NU = 5000
NI = 5000
H = 256
E = 160000
LAYERS = 3
PADU = 5120
S_UI = 2 * PADU
S_UU = PADU
LANES = 16
NW = 32
CHUNK = 64
NCHK = E // CHUNK
R_UI = S_UI // NW
R_UU = S_UU // NW
NCH = 16


def _mlp_pallas(feats, W1, b1, W2, b2):
    def body(x_ref, w1_ref, b1_ref, w2_ref, b2_ref, o_ref):
        x = x_ref[...]
        h = jnp.dot(x, w1_ref[0], preferred_element_type=F32) + b1_ref[0, 0][None, :]
        h = jnp.maximum(h, 0.0)
        o_ref[...] = jnp.dot(h, w2_ref[0], preferred_element_type=F32) + b2_ref[0, 0][None, :]

    nb = PADU // 512
    return pl.pallas_call(
        body,
        grid=(2, nb),
        in_specs=[
            pl.BlockSpec((512, 512), lambda i, j: (i * nb + j, 0)),
            pl.BlockSpec((1, 512, H), lambda i, j: (i, 0, 0)),
            pl.BlockSpec((1, 1, H), lambda i, j: (i, 0, 0)),
            pl.BlockSpec((1, H, H), lambda i, j: (i, 0, 0)),
            pl.BlockSpec((1, 1, H), lambda i, j: (i, 0, 0)),
        ],
        out_specs=pl.BlockSpec((512, H), lambda i, j: (i * nb + j, 0)),
        out_shape=jax.ShapeDtypeStruct((2 * PADU, H), F32),
    )(feats, W1, b1, W2, b2)


def _make_prop_kernel(r_rows, s_rows, scale, with_deg):
    a_rows = r_rows + 8
    mesh = plsc.VectorSubcoreMesh(core_axis_name="c", subcore_axis_name="s")

    out_type = [jax.ShapeDtypeStruct((s_rows, H), F32),
                jax.ShapeDtypeStruct((s_rows, H), F32)]
    scratch = [
        pltpu.VMEM((128,), I32),
        pltpu.VMEM((CHUNK,), I32),
        pltpu.VMEM((CHUNK,), I32),
        pltpu.VMEM((CHUNK,), I32),
        pltpu.VMEM((CHUNK,), I32),
        pltpu.VMEM((CHUNK + LANES,), I32),
        pltpu.VMEM((CHUNK, H), F32),
        pltpu.VMEM((CHUNK, H), F32),
        pltpu.VMEM((a_rows, H), F32),
        pltpu.VMEM((NCH, H), F32),
        pltpu.VMEM((NCH, LANES), F32),
        pltpu.SemaphoreType.DMA,
        pltpu.SemaphoreType.DMA,
    ]
    if with_deg:
        out_type.append(jax.ShapeDtypeStruct((s_rows, LANES), F32))
        scratch.append(pltpu.VMEM((a_rows * LANES,), F32))

    @functools.partial(pl.kernel, out_type=tuple(out_type), mesh=mesh,
                       scratch_types=scratch)
    def k(*refs):
        if with_deg:
            (emb_hbm, acc_hbm, srcs_hbm, dstl_hbm, own_hbm, bnd_hbm,
             out_hbm, accn_hbm, inv_hbm,
             bnd_v, src_v0, src_v1, dst_v, own_v, midx_v, rows_v0, rows_v1,
             acc_v, accp_v, inv_v, sem0, sem1, deg_v) = refs
        else:
            (emb_hbm, acc_hbm, srcs_hbm, dstl_hbm, own_hbm, bnd_hbm, inv_hbm,
             out_hbm, accn_hbm,
             bnd_v, src_v0, src_v1, dst_v, own_v, midx_v, rows_v0, rows_v1,
             acc_v, accp_v, inv_v, sem0, sem1) = refs
        c = lax.axis_index("c")
        s = lax.axis_index("s")
        w = c * 16 + s

        def fill_zero(r, _):
            for jj in range(H // LANES):
                acc_v[r, pl.ds(jj * LANES, LANES)] = jnp.zeros((LANES,), F32)
            if with_deg:
                deg_v[pl.ds(r * LANES, LANES)] = jnp.zeros((LANES,), F32)
            return 0

        lax.fori_loop(0, a_rows, fill_zero, 0)

        pltpu.sync_copy(bnd_hbm, bnd_v)
        lo = bnd_v[pl.ds(w, 16)][0]
        hi = bnd_v[pl.ds(w + 32, 16)][0]

        ones16 = jnp.full((LANES,), 1.0, F32)

        @pl.when(lo < hi)
        def _():
            pltpu.sync_copy(srcs_hbm.at[lo], src_v0)
            pltpu.async_copy(emb_hbm.at[src_v0], rows_v0, sem0)

        def process(j, src_v, rows_v, sem, o_src, o_rows, o_sem):
            pltpu.sync_copy(dstl_hbm.at[j], dst_v)
            pltpu.sync_copy(own_hbm.at[j], own_v)

            def group(g, _):
                ovec = own_v[pl.ds(g * LANES, LANES)]
                dvec = dst_v[pl.ds(g * LANES, LANES)]
                midx_v[pl.ds(g * LANES, LANES)] = jnp.where(ovec == w, dvec, r_rows)
                return 0

            lax.fori_loop(0, CHUNK // LANES, group, 0)
            pltpu.make_async_copy(emb_hbm.at[src_v], rows_v, sem).wait()

            @pl.when(j + 1 < hi)
            def _():
                pltpu.sync_copy(srcs_hbm.at[j + 1], o_src)
                pltpu.async_copy(emb_hbm.at[o_src], o_rows, o_sem)

            def edge4(q, _):
                mv = midx_v[pl.ds(q * 4, LANES)]
                for kk in range(4):
                    d = mv[kk]
                    e = q * 4 + kk
                    if with_deg:
                        plsc.addupdate(deg_v.at[pl.ds(d * LANES, LANES)], ones16)
                    for jj in range(H // LANES):
                        sl = pl.ds(jj * LANES, LANES)
                        plsc.addupdate(acc_v.at[d, sl], rows_v[e, sl])
                return 0

            lax.fori_loop(0, CHUNK // 4, edge4, 0)

        def pair(p, _):
            j0 = lo + 2 * p

            @pl.when(j0 < hi)
            def _():
                process(j0, src_v0, rows_v0, sem0, src_v1, rows_v1, sem1)

            @pl.when(j0 + 1 < hi)
            def _():
                process(j0 + 1, src_v1, rows_v1, sem1, src_v0, rows_v0, sem0)

            return 0

        lax.fori_loop(0, (hi - lo + 1) // 2, pair, 0)

        for t in range(r_rows // NCH):
            gbase = w * r_rows + t * NCH
            if with_deg:
                def mkinv(r, _):
                    v = deg_v[pl.ds((t * NCH + r) * LANES, LANES)]
                    inv_v[r, :] = 1.0 / jnp.maximum(v, 1.0)
                    return 0

                lax.fori_loop(0, NCH, mkinv, 0)
            else:
                pltpu.sync_copy(inv_hbm.at[pl.ds(gbase, NCH)], inv_v)
            pltpu.sync_copy(acc_hbm.at[pl.ds(gbase, NCH)], accp_v)

            def nrm(r, _):
                iv = inv_v[r, :]
                for jj in range(H // LANES):
                    sl = pl.ds(jj * LANES, LANES)
                    o = acc_v[t * NCH + r, sl] * iv
                    acc_v[t * NCH + r, sl] = o
                    accp_v[r, sl] = (accp_v[r, sl] + o) * scale
                return 0

            lax.fori_loop(0, NCH, nrm, 0)
            pltpu.sync_copy(acc_v.at[pl.ds(t * NCH, NCH)],
                            out_hbm.at[pl.ds(gbase, NCH)])
            pltpu.sync_copy(accp_v, accn_hbm.at[pl.ds(gbase, NCH)])
            if with_deg:
                pltpu.sync_copy(inv_v, inv_hbm.at[pl.ds(gbase, NCH)])

    return k


def _prep_edges(edge_index, remap_gap_at, r_rows):
    src = edge_index[0].astype(I32)
    dst = edge_index[1].astype(I32)
    if remap_gap_at is not None:
        src = src + jnp.where(src >= remap_gap_at, 120, 0).astype(I32)
        dst = dst + jnp.where(dst >= remap_gap_at, 120, 0).astype(I32)
    order = jnp.argsort(dst)
    ds = dst[order]
    ss = src[order]
    owner = ds // r_rows
    dstloc = ds - owner * r_rows
    bnd = jnp.searchsorted(ds, jnp.arange(NW + 1, dtype=I32) * r_rows).astype(I32)
    clo = bnd[:NW] // CHUNK
    chi = (bnd[1:] + CHUNK - 1) // CHUNK
    bnds = jnp.concatenate([clo, chi, jnp.zeros((128 - 2 * NW,), I32)]).astype(I32)
    return (ss.reshape(NCHK, CHUNK), dstloc.reshape(NCHK, CHUNK),
            owner.reshape(NCHK, CHUNK), bnds)


def kernel(user_feat, item_feat, W1u, b1u, W2u, b2u, W1i, b1i, W2i, b2i,
           ui_edge_index, uu_edge_index):
    feats = jnp.concatenate([
        jnp.pad(user_feat, ((0, PADU - NU), (0, 0))),
        jnp.pad(item_feat, ((0, PADU - NI), (0, 0))),
    ], axis=0)
    W1 = jnp.stack([W1u, W1i])
    b1 = jnp.stack([b1u, b1i])[:, None, :]
    W2 = jnp.stack([W2u, W2i])
    b2 = jnp.stack([b2u, b2i])[:, None, :]
    ego = _mlp_pallas(feats, W1, b1, W2, b2)

    ui = _prep_edges(ui_edge_index, NU, R_UI)
    uu = _prep_edges(uu_edge_index, None, R_UU)

    out_i, acc_i = ego, ego
    inv_ui = None
    for layer in range(LAYERS):
        scale = 1.0 / (LAYERS + 1.0) if layer == LAYERS - 1 else 1.0
        prop = _make_prop_kernel(R_UI, S_UI, scale, with_deg=(layer == 0))
        if layer == 0:
            out_i, acc_i, inv_ui = prop(out_i, acc_i, *ui)
        else:
            out_i, acc_i = prop(out_i, acc_i, *ui[:3], ui[3], inv_ui)

    ego_u = ego[:PADU]
    out_u, acc_u = ego_u, ego_u
    inv_uu = None
    for layer in range(LAYERS):
        scale = 1.0 / (LAYERS + 1.0) if layer == LAYERS - 1 else 1.0
        prop = _make_prop_kernel(R_UU, S_UU, scale, with_deg=(layer == 0))
        if layer == 0:
            out_u, acc_u, inv_uu = prop(out_u, acc_u, *uu)
        else:
            out_u, acc_u = prop(out_u, acc_u, *uu[:3], uu[3], inv_uu)

    user_final = acc_u[:NU]
    item_emb = acc_i[PADU:PADU + NI]
    return (user_final, item_emb)


# speedup vs baseline: 1.5350x; 1.5350x over previous
import functools

import jax
import jax.numpy as jnp
from jax import lax
from jax.experimental import pallas as pl
from jax.experimental.pallas import tpu as pltpu
from jax.experimental.pallas import tpu_sc as plsc

F32 = jnp.float32
I32 = jnp.int32

NU = 5000
NI = 5000
H = 256
E = 160000
LAYERS = 3
PADU = 5120
S_UI = 2 * PADU
S_UU = PADU
LANES = 16
NW = 32
CHUNK = 64
NCHK = E // CHUNK
R_UI = S_UI // NW
R_UU = S_UU // NW
NCH = 16


def _mlp_pallas(feats, W1, b1, W2, b2):
    def body(x_ref, w1_ref, b1_ref, w2_ref, b2_ref, o_ref):
        x = x_ref[...]
        h = jnp.dot(x, w1_ref[0], preferred_element_type=F32) + b1_ref[0, 0][None, :]
        h = jnp.maximum(h, 0.0)
        o_ref[...] = jnp.dot(h, w2_ref[0], preferred_element_type=F32) + b2_ref[0, 0][None, :]

    nb = PADU // 512
    return pl.pallas_call(
        body,
        grid=(2, nb),
        in_specs=[
            pl.BlockSpec((512, 512), lambda i, j: (i * nb + j, 0)),
            pl.BlockSpec((1, 512, H), lambda i, j: (i, 0, 0)),
            pl.BlockSpec((1, 1, H), lambda i, j: (i, 0, 0)),
            pl.BlockSpec((1, H, H), lambda i, j: (i, 0, 0)),
            pl.BlockSpec((1, 1, H), lambda i, j: (i, 0, 0)),
        ],
        out_specs=pl.BlockSpec((512, H), lambda i, j: (i * nb + j, 0)),
        out_shape=jax.ShapeDtypeStruct((2 * PADU, H), F32),
    )(feats, W1, b1, W2, b2)


def _make_prop_kernel(r_rows, s_rows, scale, with_deg):
    a_rows = r_rows + 8
    mesh = plsc.VectorSubcoreMesh(core_axis_name="c", subcore_axis_name="s")

    out_type = [jax.ShapeDtypeStruct((s_rows * H,), F32),
                jax.ShapeDtypeStruct((s_rows * H,), F32)]
    scratch = [
        pltpu.VMEM((128,), I32),
        pltpu.VMEM((CHUNK,), I32),
        pltpu.VMEM((CHUNK,), I32),
        pltpu.VMEM((CHUNK,), I32),
        pltpu.VMEM((CHUNK,), I32),
        pltpu.VMEM((CHUNK + LANES,), I32),
        pltpu.VMEM((CHUNK + LANES,), I32),
        pltpu.VMEM((CHUNK, H), F32),
        pltpu.VMEM((CHUNK, H), F32),
        pltpu.VMEM((a_rows * H,), F32),
        pltpu.VMEM((NCH * H,), F32),
        pltpu.VMEM((NCH, LANES), F32),
        pltpu.SemaphoreType.DMA,
        pltpu.SemaphoreType.DMA,
    ]
    if with_deg:
        out_type.append(jax.ShapeDtypeStruct((s_rows, LANES), F32))
        scratch.append(pltpu.VMEM((a_rows * LANES,), F32))

    @functools.partial(pl.kernel, out_type=tuple(out_type), mesh=mesh,
                       scratch_types=scratch)
    def k(*refs):
        if with_deg:
            (emb_hbm, acc_hbm, srcs_hbm, dstl_hbm, own_hbm, segs_hbm, bnd_hbm,
             out_hbm, accn_hbm, inv_hbm,
             bnd_v, src_v0, src_v1, dst_v, own_v, midx_v, segc_v, rows_v0,
             rows_v1, acc_v, accp_v, inv_v, sem0, sem1, deg_v) = refs
        else:
            (emb_hbm, acc_hbm, srcs_hbm, dstl_hbm, own_hbm, segs_hbm, bnd_hbm,
             inv_hbm, out_hbm, accn_hbm,
             bnd_v, src_v0, src_v1, dst_v, own_v, midx_v, segc_v, rows_v0,
             rows_v1, acc_v, accp_v, inv_v, sem0, sem1) = refs
        c = lax.axis_index("c")
        s = lax.axis_index("s")
        w = c * 16 + s

        def fill_zero(r, _):
            acc_v[pl.ds(r * LANES, LANES)] = jnp.zeros((LANES,), F32)
            return 0

        lax.fori_loop(0, a_rows * H // LANES, fill_zero, 0)
        if with_deg:
            def fill_zero_deg(r, _):
                deg_v[pl.ds(r * LANES, LANES)] = jnp.zeros((LANES,), F32)
                return 0

            lax.fori_loop(0, a_rows, fill_zero_deg, 0)

        pltpu.sync_copy(bnd_hbm, bnd_v)
        lo = bnd_v[pl.ds(w, 16)][0]
        hi = bnd_v[pl.ds(w + 32, 16)][0]

        ones16 = jnp.full((LANES,), 1.0, F32)

        @pl.when(lo < hi)
        def _():
            pltpu.sync_copy(srcs_hbm.at[lo], src_v0)
            pltpu.async_copy(emb_hbm.at[src_v0], rows_v0, sem0)

        def process(j, src_v, rows_v, sem, o_src, o_rows, o_sem):
            pltpu.sync_copy(dstl_hbm.at[j], dst_v)
            pltpu.sync_copy(own_hbm.at[j], own_v)
            pltpu.sync_copy(segs_hbm.at[j], segc_v)

            def group(g, _):
                ovec = own_v[pl.ds(g * LANES, LANES)]
                dvec = dst_v[pl.ds(g * LANES, LANES)]
                midx_v[pl.ds(g * LANES, LANES)] = jnp.where(ovec == w, dvec, r_rows)
                return 0

            lax.fori_loop(0, CHUNK // LANES, group, 0)
            pltpu.make_async_copy(emb_hbm.at[src_v], rows_v, sem).wait()

            @pl.when(j + 1 < hi)
            def _():
                pltpu.sync_copy(srcs_hbm.at[j + 1], o_src)
                pltpu.async_copy(emb_hbm.at[o_src], o_rows, o_sem)

            zero16 = jnp.zeros((LANES,), F32)
            nseg = segc_v[pl.ds(CHUNK, LANES)][0]

            def seg(si, eptr):
                d = midx_v[pl.ds(eptr, LANES)][0]
                cnt = segc_v[pl.ds(si, LANES)][0]

                def inner(k, vs):
                    return tuple(
                        vs[jj] + rows_v[eptr + k, pl.ds(jj * LANES, LANES)]
                        for jj in range(H // LANES))

                vs = lax.fori_loop(0, cnt, inner, (zero16,) * (H // LANES))
                if with_deg:
                    fc = lax.broadcast_in_dim(cnt.astype(F32), (LANES,), ())
                    plsc.addupdate(deg_v.at[pl.ds(d * LANES, LANES)], fc)
                for jj in range(H // LANES):
                    plsc.addupdate(acc_v.at[pl.ds(d * H + jj * LANES, LANES)],
                                   vs[jj])
                return eptr + cnt

            lax.fori_loop(0, nseg, seg, jnp.int32(0))

        def pair(p, _):
            j0 = lo + 2 * p

            @pl.when(j0 < hi)
            def _():
                process(j0, src_v0, rows_v0, sem0, src_v1, rows_v1, sem1)

            @pl.when(j0 + 1 < hi)
            def _():
                process(j0 + 1, src_v1, rows_v1, sem1, src_v0, rows_v0, sem0)

            return 0

        lax.fori_loop(0, (hi - lo + 1) // 2, pair, 0)

        for t in range(r_rows // NCH):
            gbase = w * r_rows + t * NCH
            if with_deg:
                def mkinv(r, _):
                    v = deg_v[pl.ds((t * NCH + r) * LANES, LANES)]
                    inv_v[r, :] = 1.0 / jnp.maximum(v, 1.0)
                    return 0

                lax.fori_loop(0, NCH, mkinv, 0)
            else:
                pltpu.sync_copy(inv_hbm.at[pl.ds(gbase, NCH)], inv_v)
            pltpu.sync_copy(acc_hbm.at[pl.ds(gbase * H, NCH * H)], accp_v)

            def nrm(r, _):
                iv = inv_v[r, :]
                for jj in range(H // LANES):
                    sl = pl.ds((t * NCH + r) * H + jj * LANES, LANES)
                    so = pl.ds(r * H + jj * LANES, LANES)
                    o = acc_v[sl] * iv
                    acc_v[sl] = o
                    accp_v[so] = (accp_v[so] + o) * scale
                return 0

            lax.fori_loop(0, NCH, nrm, 0)
            pltpu.sync_copy(acc_v.at[pl.ds(t * NCH * H, NCH * H)],
                            out_hbm.at[pl.ds(gbase * H, NCH * H)])
            pltpu.sync_copy(accp_v, accn_hbm.at[pl.ds(gbase * H, NCH * H)])
            if with_deg:
                pltpu.sync_copy(inv_v, inv_hbm.at[pl.ds(gbase, NCH)])

    return k


def _prep_edges(edge_index, remap_gap_at, r_rows):
    src = edge_index[0].astype(I32)
    dst = edge_index[1].astype(I32)
    if remap_gap_at is not None:
        src = src + jnp.where(src >= remap_gap_at, 120, 0).astype(I32)
        dst = dst + jnp.where(dst >= remap_gap_at, 120, 0).astype(I32)
    order = jnp.argsort(dst)
    ds = dst[order]
    ss = src[order]
    owner = ds // r_rows
    dstloc = ds - owner * r_rows
    bnd = jnp.searchsorted(ds, jnp.arange(NW + 1, dtype=I32) * r_rows).astype(I32)
    clo = bnd[:NW] // CHUNK
    chi = (bnd[1:] + CHUNK - 1) // CHUNK
    bnds = jnp.concatenate([clo, chi, jnp.zeros((128 - 2 * NW,), I32)]).astype(I32)
    ds2 = ds.reshape(NCHK, CHUNK)
    prev = jnp.concatenate([jnp.full((NCHK, 1), -1, I32), ds2[:, :-1]], axis=1)
    f = ds2 != prev
    pos = jnp.arange(CHUNK, dtype=I32)[None, :]
    big = jnp.where(f, pos, CHUNK).astype(I32)
    ns = lax.cummin(big[:, ::-1], axis=1)[:, ::-1]
    ns2 = jnp.concatenate([ns[:, 1:], jnp.full((NCHK, 1), CHUNK, I32)], axis=1)
    length = jnp.where(f, ns2 - pos, 0).astype(I32)
    order = jnp.argsort(jnp.logical_not(f), axis=1, stable=True)
    cnts = jnp.take_along_axis(length, order, axis=1)
    nseg = f.sum(axis=1, dtype=I32)
    segs = jnp.concatenate(
        [cnts, nseg[:, None], jnp.zeros((NCHK, LANES - 1), I32)], axis=1)
    return (ss.reshape(NCHK, CHUNK), dstloc.reshape(NCHK, CHUNK),
            owner.reshape(NCHK, CHUNK), segs, bnds)


def kernel(user_feat, item_feat, W1u, b1u, W2u, b2u, W1i, b1i, W2i, b2i,
           ui_edge_index, uu_edge_index):
    feats = jnp.concatenate([
        jnp.pad(user_feat, ((0, PADU - NU), (0, 0))),
        jnp.pad(item_feat, ((0, PADU - NI), (0, 0))),
    ], axis=0)
    W1 = jnp.stack([W1u, W1i])
    b1 = jnp.stack([b1u, b1i])[:, None, :]
    W2 = jnp.stack([W2u, W2i])
    b2 = jnp.stack([b2u, b2i])[:, None, :]
    ego = _mlp_pallas(feats, W1, b1, W2, b2)

    ui = _prep_edges(ui_edge_index, NU, R_UI)
    uu = _prep_edges(uu_edge_index, None, R_UU)

    out_i, acc_i = ego, ego.reshape(-1)
    inv_ui = None
    for layer in range(LAYERS):
        scale = 1.0 / (LAYERS + 1.0) if layer == LAYERS - 1 else 1.0
        prop = _make_prop_kernel(R_UI, S_UI, scale, with_deg=(layer == 0))
        if layer == 0:
            out_i, acc_i, inv_ui = prop(out_i, acc_i, *ui)
        else:
            out_i, acc_i = prop(out_i, acc_i, *ui[:4], ui[4], inv_ui)
        out_i = out_i.reshape(S_UI, H)

    ego_u = ego[:PADU]
    out_u, acc_u = ego_u, ego_u.reshape(-1)
    inv_uu = None
    for layer in range(LAYERS):
        scale = 1.0 / (LAYERS + 1.0) if layer == LAYERS - 1 else 1.0
        prop = _make_prop_kernel(R_UU, S_UU, scale, with_deg=(layer == 0))
        if layer == 0:
            out_u, acc_u, inv_uu = prop(out_u, acc_u, *uu)
        else:
            out_u, acc_u = prop(out_u, acc_u, *uu[:4], uu[4], inv_uu)
        out_u = out_u.reshape(S_UU, H)

    user_final = acc_u.reshape(S_UU, H)[:NU]
    item_emb = acc_i.reshape(S_UI, H)[PADU:PADU + NI]
    return (user_final, item_emb)
